# baseline (speedup 1.0000x reference)
import functools

import jax
import jax.numpy as jnp
from jax import lax
from jax.experimental import pallas as pl
from jax.experimental.pallas import tpu as pltpu
from jax.experimental.pallas import tpu_sc as plsc

B, C_IN, N = 2, 192, 1024
D, K = 64, 512

_NC, _NS, _L = 2, 16, 16
_NW = _NC * _NS
_TOK = B * N


def _tc_body(z_ref, w_ref, emb_ref, idx_ref, embt_ref):
    zb = z_ref[0]
    w = w_ref[...]
    emb = emb_ref[...]
    hi = lax.Precision.HIGHEST
    ze = jnp.dot(w.astype(jnp.bfloat16), zb.astype(jnp.bfloat16),
                 preferred_element_type=jnp.float32)
    scores = jnp.dot(emb, ze, preferred_element_type=jnp.float32,
                     precision=hi)
    esq2 = 0.5 * jnp.sum(emb * emb, axis=1, keepdims=True)
    dist = esq2 - scores

    iota = lax.broadcasted_iota(jnp.int32, (K, N), 0)
    m1 = jnp.min(dist, axis=0, keepdims=True)
    i1 = jnp.min(jnp.where(dist == m1, iota, K), axis=0, keepdims=True)
    dist2 = jnp.where(iota == i1, jnp.float32(jnp.inf), dist)
    m2 = jnp.min(dist2, axis=0, keepdims=True)
    i2 = jnp.min(jnp.where(dist2 == m2, iota, K), axis=0, keepdims=True)

    oh1 = (iota == i1).astype(jnp.float32)
    oh2 = (iota == i2).astype(jnp.float32)
    dn = (((0,), (0,)), ((), ()))
    e1 = lax.dot_general(emb, oh1, dn, precision=hi,
                         preferred_element_type=jnp.float32)
    e2 = lax.dot_general(emb, oh2, dn, precision=hi,
                         preferred_element_type=jnp.float32)
    d1 = jnp.sum((ze - e1) ** 2, axis=0, keepdims=True)
    d2 = jnp.sum((ze - e2) ** 2, axis=0, keepdims=True)
    pick2 = (d2 < d1) | ((d2 == d1) & (i2 < i1))
    idx_ref[0] = jnp.where(pick2, i2, i1)

    @pl.when(pl.program_id(0) == 0)
    def _():
        embt_ref[...] = emb.T


_tc_call = pl.pallas_call(
    _tc_body,
    grid=(B,),
    in_specs=[
        pl.BlockSpec((1, C_IN, N), lambda b: (b, 0, 0)),
        pl.BlockSpec((D, C_IN), lambda b: (0, 0)),
        pl.BlockSpec((K, D), lambda b: (0, 0)),
    ],
    out_specs=[
        pl.BlockSpec((1, 1, N), lambda b: (b, 0, 0)),
        pl.BlockSpec((D, K), lambda b: (0, 0)),
    ],
    out_shape=[
        jax.ShapeDtypeStruct((B, 1, N), jnp.int32),
        jax.ShapeDtypeStruct((D, K), jnp.float32),
    ],
)


@functools.cache
def _make_sc_gather():
    mesh = plsc.VectorSubcoreMesh(core_axis_name="c", subcore_axis_name="s")

    tblk = 128
    drows = D // 2

    @functools.partial(
        pl.kernel,
        mesh=mesh,
        compiler_params=pltpu.CompilerParams(needs_layout_passes=False),
        out_type=jax.ShapeDtypeStruct((B, D, N), jnp.float32),
        scratch_types=[
            pltpu.VMEM((tblk,), jnp.int32),
            pltpu.VMEM((drows, K), jnp.float32),
            pltpu.VMEM((drows, tblk), jnp.float32),
            pltpu.SemaphoreType.DMA,
        ],
    )
    def _sc_gather(embt_hbm, idx_hbm, out_hbm, idx_v, embt_v, outt_v, sem):
        wid = lax.axis_index("s") * _NC + lax.axis_index("c")
        b = wid // (_NW // B)
        r = wid % (_NW // B)
        dh = r // (N // tblk)
        tb = r % (N // tblk)
        pltpu.sync_copy(embt_hbm.at[pl.ds(dh * drows, drows), :], embt_v)
        pltpu.sync_copy(idx_hbm.at[b, 0, pl.ds(tb * tblk, tblk)], idx_v)

        def row(dl, _):
            drow = jnp.full((_L,), dl, jnp.int32)
            for g in range(tblk // _L):
                tok = idx_v[pl.ds(g * _L, _L)]
                outt_v[dl, pl.ds(g * _L, _L)] = plsc.load_gather(
                    embt_v, [drow, tok])
            return 0

        lax.fori_loop(0, drows, row, 0, unroll=2)
        pltpu.sync_copy(
            outt_v, out_hbm.at[b, pl.ds(dh * drows, drows),
                               pl.ds(tb * tblk, tblk)])

    return _sc_gather


def kernel(z, W, emb):
    idx, embt = _tc_call(z, W, emb)
    return _make_sc_gather()(embt, idx)


# speedup vs baseline: 1.0627x; 1.0627x over previous
import functools

import jax
import jax.numpy as jnp
from jax import lax
from jax.experimental import pallas as pl
from jax.experimental.pallas import tpu as pltpu
from jax.experimental.pallas import tpu_sc as plsc

B, C_IN, N = 2, 192, 1024
D, K = 64, 512

_NC, _NS, _L = 2, 16, 16
_NW = _NC * _NS
_TOK = B * N


def _tc_body(z_ref, w_ref, emb_ref, idx_ref, embt_ref):
    zb = z_ref[0]
    w = w_ref[...]
    emb = emb_ref[...]
    hi = lax.Precision.HIGHEST
    ze = jnp.dot(w.astype(jnp.bfloat16), zb.astype(jnp.bfloat16),
                 preferred_element_type=jnp.float32)
    scores = jnp.dot(emb, ze, preferred_element_type=jnp.float32,
                     precision=hi)
    esq2 = 0.5 * jnp.sum(emb * emb, axis=1, keepdims=True)
    dist = esq2 - scores

    iota = lax.broadcasted_iota(jnp.int32, (K, N), 0)
    m1 = jnp.min(dist, axis=0, keepdims=True)
    i1 = jnp.min(jnp.where(dist == m1, iota, K), axis=0, keepdims=True)
    dist2 = jnp.where(iota == i1, jnp.float32(jnp.inf), dist)
    m2 = jnp.min(dist2, axis=0, keepdims=True)
    i2 = jnp.min(jnp.where(dist2 == m2, iota, K), axis=0, keepdims=True)

    oh1 = (iota == i1).astype(jnp.float32)
    oh2 = (iota == i2).astype(jnp.float32)
    dn = (((0,), (0,)), ((), ()))
    e1 = lax.dot_general(emb, oh1, dn, precision=hi,
                         preferred_element_type=jnp.float32)
    e2 = lax.dot_general(emb, oh2, dn, precision=hi,
                         preferred_element_type=jnp.float32)
    d1 = jnp.sum((ze - e1) ** 2, axis=0, keepdims=True)
    d2 = jnp.sum((ze - e2) ** 2, axis=0, keepdims=True)
    pick2 = (d2 < d1) | ((d2 == d1) & (i2 < i1))
    idx_ref[0] = jnp.where(pick2, i2, i1)

    @pl.when(pl.program_id(0) == 0)
    def _():
        embt_ref[...] = emb.T


_tc_call = pl.pallas_call(
    _tc_body,
    grid=(B,),
    in_specs=[
        pl.BlockSpec((1, C_IN, N), lambda b: (b, 0, 0)),
        pl.BlockSpec((D, C_IN), lambda b: (0, 0)),
        pl.BlockSpec((K, D), lambda b: (0, 0)),
    ],
    out_specs=[
        pl.BlockSpec((1, 1, N), lambda b: (b, 0, 0)),
        pl.BlockSpec((D, K), lambda b: (0, 0)),
    ],
    out_shape=[
        jax.ShapeDtypeStruct((B, 1, N), jnp.int32),
        jax.ShapeDtypeStruct((D, K), jnp.float32),
    ],
)


@functools.cache
def _make_sc_gather():
    mesh = plsc.VectorSubcoreMesh(core_axis_name="c", subcore_axis_name="s")

    tblk = 128
    drows = D // 2

    @functools.partial(
        pl.kernel,
        mesh=mesh,
        compiler_params=pltpu.CompilerParams(needs_layout_passes=False),
        out_type=jax.ShapeDtypeStruct((B, D, N), jnp.float32),
        scratch_types=[
            pltpu.VMEM((tblk,), jnp.int32),
            pltpu.VMEM((drows, K), jnp.float32),
            pltpu.VMEM((drows, tblk), jnp.float32),
            pltpu.SemaphoreType.DMA,
        ],
    )
    def _sc_gather(embt_hbm, idx_hbm, out_hbm, idx_v, embt_v, outt_v, sem):
        wid = lax.axis_index("s") * _NC + lax.axis_index("c")
        b = wid // (_NW // B)
        r = wid % (_NW // B)
        dh = r // (N // tblk)
        tb = r % (N // tblk)
        pltpu.sync_copy(embt_hbm.at[pl.ds(dh * drows, drows), :], embt_v)
        pltpu.sync_copy(idx_hbm.at[b, 0, pl.ds(tb * tblk, tblk)], idx_v)

        @plsc.parallel_loop(0, drows, unroll=4)
        def _row(dl):
            drow = jnp.full((_L,), dl, jnp.int32)
            for g in range(tblk // _L):
                tok = idx_v[pl.ds(g * _L, _L)]
                outt_v[dl, pl.ds(g * _L, _L)] = plsc.load_gather(
                    embt_v, [drow, tok])
        pltpu.sync_copy(
            outt_v, out_hbm.at[b, pl.ds(dh * drows, drows),
                               pl.ds(tb * tblk, tblk)])

    return _sc_gather


def kernel(z, W, emb):
    idx, embt = _tc_call(z, W, emb)
    return _make_sc_gather()(embt, idx)


# speedup vs baseline: 1.1516x; 1.0836x over previous
import functools

import jax
import jax.numpy as jnp
from jax import lax
from jax.experimental import pallas as pl
from jax.experimental.pallas import tpu as pltpu
from jax.experimental.pallas import tpu_sc as plsc

B, C_IN, N = 2, 192, 1024
D, K = 64, 512

_NC, _NS, _L = 2, 16, 16
_NW = _NC * _NS
_TOK = B * N


def _tc_body(z_ref, w_ref, emb_ref, idx_ref, embt_ref):
    zb = z_ref[0]
    w = w_ref[...]
    emb = emb_ref[...]
    hi = lax.Precision.HIGHEST
    ze = jnp.dot(w.astype(jnp.bfloat16), zb.astype(jnp.bfloat16),
                 preferred_element_type=jnp.float32)
    scores = jnp.dot(emb, ze, preferred_element_type=jnp.float32,
                     precision=hi)
    esq2 = 0.5 * jnp.sum(emb * emb, axis=1, keepdims=True)
    dist = esq2 - scores

    iota = lax.broadcasted_iota(jnp.int32, (K, N), 0)
    m1 = jnp.min(dist, axis=0, keepdims=True)
    i1 = jnp.min(jnp.where(dist == m1, iota, K), axis=0, keepdims=True)
    dist2 = jnp.where(iota == i1, jnp.float32(jnp.inf), dist)
    m2 = jnp.min(dist2, axis=0, keepdims=True)
    i2 = jnp.min(jnp.where(dist2 == m2, iota, K), axis=0, keepdims=True)

    oh1 = (iota == i1).astype(jnp.bfloat16)
    oh2 = (iota == i2).astype(jnp.bfloat16)
    dn = (((0,), (0,)), ((), ()))
    bh = emb.astype(jnp.bfloat16)
    r1 = emb - bh.astype(jnp.float32)
    b1 = r1.astype(jnp.bfloat16)
    b2 = (r1 - b1.astype(jnp.float32)).astype(jnp.bfloat16)

    def _sel(oh):
        p0 = lax.dot_general(bh, oh, dn, preferred_element_type=jnp.float32)
        p1 = lax.dot_general(b1, oh, dn, preferred_element_type=jnp.float32)
        p2 = lax.dot_general(b2, oh, dn, preferred_element_type=jnp.float32)
        return (p0 + p1) + p2

    e1 = _sel(oh1)
    e2 = _sel(oh2)
    d1 = jnp.sum((ze - e1) ** 2, axis=0, keepdims=True)
    d2 = jnp.sum((ze - e2) ** 2, axis=0, keepdims=True)
    pick2 = (d2 < d1) | ((d2 == d1) & (i2 < i1))
    idx_ref[0] = jnp.where(pick2, i2, i1)

    @pl.when(pl.program_id(0) == 0)
    def _():
        embt_ref[...] = emb.T


_tc_call = pl.pallas_call(
    _tc_body,
    grid=(B,),
    in_specs=[
        pl.BlockSpec((1, C_IN, N), lambda b: (b, 0, 0)),
        pl.BlockSpec((D, C_IN), lambda b: (0, 0)),
        pl.BlockSpec((K, D), lambda b: (0, 0)),
    ],
    out_specs=[
        pl.BlockSpec((1, 1, N), lambda b: (b, 0, 0)),
        pl.BlockSpec((D, K), lambda b: (0, 0)),
    ],
    out_shape=[
        jax.ShapeDtypeStruct((B, 1, N), jnp.int32),
        jax.ShapeDtypeStruct((D, K), jnp.float32),
    ],
)


@functools.cache
def _make_sc_gather():
    mesh = plsc.VectorSubcoreMesh(core_axis_name="c", subcore_axis_name="s")

    tblk = 128
    drows = D // 2

    @functools.partial(
        pl.kernel,
        mesh=mesh,
        compiler_params=pltpu.CompilerParams(needs_layout_passes=False),
        out_type=jax.ShapeDtypeStruct((B, D, N), jnp.float32),
        scratch_types=[
            pltpu.VMEM((tblk,), jnp.int32),
            pltpu.VMEM((drows, K), jnp.float32),
            pltpu.VMEM((drows, tblk), jnp.float32),
            pltpu.SemaphoreType.DMA,
        ],
    )
    def _sc_gather(embt_hbm, idx_hbm, out_hbm, idx_v, embt_v, outt_v, sem):
        wid = lax.axis_index("s") * _NC + lax.axis_index("c")
        b = wid // (_NW // B)
        r = wid % (_NW // B)
        dh = r // (N // tblk)
        tb = r % (N // tblk)
        pltpu.sync_copy(embt_hbm.at[pl.ds(dh * drows, drows), :], embt_v)
        pltpu.sync_copy(idx_hbm.at[b, 0, pl.ds(tb * tblk, tblk)], idx_v)

        @plsc.parallel_loop(0, drows, unroll=4)
        def _row(dl):
            drow = jnp.full((_L,), dl, jnp.int32)
            for g in range(tblk // _L):
                tok = idx_v[pl.ds(g * _L, _L)]
                outt_v[dl, pl.ds(g * _L, _L)] = plsc.load_gather(
                    embt_v, [drow, tok])
        pltpu.sync_copy(
            outt_v, out_hbm.at[b, pl.ds(dh * drows, drows),
                               pl.ds(tb * tblk, tblk)])

    return _sc_gather


def kernel(z, W, emb):
    idx, embt = _tc_call(z, W, emb)
    return _make_sc_gather()(embt, idx)


# speedup vs baseline: 1.1657x; 1.0122x over previous
import functools

import jax
import jax.numpy as jnp
from jax import lax
from jax.experimental import pallas as pl
from jax.experimental.pallas import tpu as pltpu
from jax.experimental.pallas import tpu_sc as plsc

B, C_IN, N = 2, 192, 1024
D, K = 64, 512

_NC, _NS, _L = 2, 16, 16
_NW = _NC * _NS
_TOK = B * N


_W = B * N


def _tc_body(z_ref, w_ref, emb_ref, idx_ref, embt_ref):
    w = w_ref[...]
    emb = emb_ref[...]
    hi = lax.Precision.HIGHEST
    wb = w.astype(jnp.bfloat16)
    zb = jnp.concatenate([z_ref[0], z_ref[1]], axis=1)
    ze = jnp.dot(wb, zb.astype(jnp.bfloat16),
                 preferred_element_type=jnp.float32)
    scores = jnp.dot(emb, ze, preferred_element_type=jnp.float32,
                     precision=hi)
    esq2 = 0.5 * jnp.sum(emb * emb, axis=1, keepdims=True)
    dist = esq2 - scores

    iota = lax.broadcasted_iota(jnp.int32, (K, _W), 0)
    m1 = jnp.min(dist, axis=0, keepdims=True)
    i1 = jnp.min(jnp.where(dist == m1, iota, K), axis=0, keepdims=True)
    dist2 = jnp.where(iota == i1, jnp.float32(jnp.inf), dist)
    m2 = jnp.min(dist2, axis=0, keepdims=True)
    i2 = jnp.min(jnp.where(dist2 == m2, iota, K), axis=0, keepdims=True)

    oh1 = (iota == i1).astype(jnp.bfloat16)
    oh2 = (iota == i2).astype(jnp.bfloat16)
    dn = (((0,), (0,)), ((), ()))
    bh = emb.astype(jnp.bfloat16)
    r1 = emb - bh.astype(jnp.float32)
    b1 = r1.astype(jnp.bfloat16)
    b2 = (r1 - b1.astype(jnp.float32)).astype(jnp.bfloat16)

    def _sel(oh):
        p0 = lax.dot_general(bh, oh, dn, preferred_element_type=jnp.float32)
        p1 = lax.dot_general(b1, oh, dn, preferred_element_type=jnp.float32)
        p2 = lax.dot_general(b2, oh, dn, preferred_element_type=jnp.float32)
        return (p0 + p1) + p2

    e1 = _sel(oh1)
    e2 = _sel(oh2)
    d1 = jnp.sum((ze - e1) ** 2, axis=0, keepdims=True)
    d2 = jnp.sum((ze - e2) ** 2, axis=0, keepdims=True)
    pick2 = (d2 < d1) | ((d2 == d1) & (i2 < i1))
    idx_ref[...] = jnp.where(pick2, i2, i1)

    embt_ref[...] = emb.T


_tc_call = pl.pallas_call(
    _tc_body,
    in_specs=[
        pl.BlockSpec((B, C_IN, N), lambda: (0, 0, 0)),
        pl.BlockSpec((D, C_IN), lambda: (0, 0)),
        pl.BlockSpec((K, D), lambda: (0, 0)),
    ],
    out_specs=[
        pl.BlockSpec((1, _W), lambda: (0, 0)),
        pl.BlockSpec((D, K), lambda: (0, 0)),
    ],
    out_shape=[
        jax.ShapeDtypeStruct((1, _W), jnp.int32),
        jax.ShapeDtypeStruct((D, K), jnp.float32),
    ],
)


@functools.cache
def _make_sc_gather():
    mesh = plsc.VectorSubcoreMesh(core_axis_name="c", subcore_axis_name="s")

    tblk = 128
    drows = D // 2

    @functools.partial(
        pl.kernel,
        mesh=mesh,
        compiler_params=pltpu.CompilerParams(needs_layout_passes=False),
        out_type=jax.ShapeDtypeStruct((B, D, N), jnp.float32),
        scratch_types=[
            pltpu.VMEM((tblk,), jnp.int32),
            pltpu.VMEM((drows, K), jnp.float32),
            pltpu.VMEM((drows, tblk), jnp.float32),
            pltpu.SemaphoreType.DMA,
        ],
    )
    def _sc_gather(embt_hbm, idx_hbm, out_hbm, idx_v, embt_v, outt_v, sem):
        wid = lax.axis_index("s") * _NC + lax.axis_index("c")
        b = wid // (_NW // B)
        r = wid % (_NW // B)
        dh = r // (N // tblk)
        tb = r % (N // tblk)
        pltpu.sync_copy(embt_hbm.at[pl.ds(dh * drows, drows), :], embt_v)
        pltpu.sync_copy(idx_hbm.at[pl.ds(b * N + tb * tblk, tblk)], idx_v)

        @plsc.parallel_loop(0, drows, unroll=4)
        def _row(dl):
            drow = jnp.full((_L,), dl, jnp.int32)
            for g in range(tblk // _L):
                tok = idx_v[pl.ds(g * _L, _L)]
                outt_v[dl, pl.ds(g * _L, _L)] = plsc.load_gather(
                    embt_v, [drow, tok])
        pltpu.sync_copy(
            outt_v, out_hbm.at[b, pl.ds(dh * drows, drows),
                               pl.ds(tb * tblk, tblk)])

    return _sc_gather


def kernel(z, W, emb):
    idx, embt = _tc_call(z, W, emb)
    return _make_sc_gather()(embt, idx.reshape(_TOK))


# speedup vs baseline: 1.2038x; 1.0327x over previous
import functools

import jax
import jax.numpy as jnp
from jax import lax
from jax.experimental import pallas as pl
from jax.experimental.pallas import tpu as pltpu
from jax.experimental.pallas import tpu_sc as plsc

B, C_IN, N = 2, 192, 1024
D, K = 64, 512

_NC, _NS, _L = 2, 16, 16
_NW = _NC * _NS
_TOK = B * N


_W = B * N


def _tc_body(z_ref, w_ref, emb_ref, idx_ref, embt_ref):
    w = w_ref[...]
    emb = emb_ref[...]
    hi = lax.Precision.HIGHEST
    wb = w.astype(jnp.bfloat16)
    zb = jnp.concatenate([z_ref[0], z_ref[1]], axis=1)
    ze = jnp.dot(wb, zb.astype(jnp.bfloat16),
                 preferred_element_type=jnp.float32)
    scores = jnp.dot(emb, ze, preferred_element_type=jnp.float32,
                     precision=hi)
    esq2 = 0.5 * jnp.sum(emb * emb, axis=1, keepdims=True)
    dist = esq2 - scores

    iota = lax.broadcasted_iota(jnp.int32, (K, _W), 0)
    i1 = jnp.argmin(dist, axis=0, keepdims=True).astype(jnp.int32)
    dist2 = jnp.where(iota == i1, jnp.float32(jnp.inf), dist)
    i2 = jnp.argmin(dist2, axis=0, keepdims=True).astype(jnp.int32)

    oh1 = (iota == i1).astype(jnp.bfloat16)
    oh2 = (iota == i2).astype(jnp.bfloat16)
    dn = (((0,), (0,)), ((), ()))
    bh = emb.astype(jnp.bfloat16)
    r1 = emb - bh.astype(jnp.float32)
    b1 = r1.astype(jnp.bfloat16)
    b2 = (r1 - b1.astype(jnp.float32)).astype(jnp.bfloat16)

    def _sel(oh):
        p0 = lax.dot_general(bh, oh, dn, preferred_element_type=jnp.float32)
        p1 = lax.dot_general(b1, oh, dn, preferred_element_type=jnp.float32)
        p2 = lax.dot_general(b2, oh, dn, preferred_element_type=jnp.float32)
        return (p0 + p1) + p2

    e1 = _sel(oh1)
    e2 = _sel(oh2)
    d1 = jnp.sum((ze - e1) ** 2, axis=0, keepdims=True)
    d2 = jnp.sum((ze - e2) ** 2, axis=0, keepdims=True)
    pick2 = (d2 < d1) | ((d2 == d1) & (i2 < i1))
    idx_ref[...] = jnp.where(pick2, i2, i1)

    embt_ref[...] = emb.T


_tc_call = pl.pallas_call(
    _tc_body,
    in_specs=[
        pl.BlockSpec((B, C_IN, N), lambda: (0, 0, 0)),
        pl.BlockSpec((D, C_IN), lambda: (0, 0)),
        pl.BlockSpec((K, D), lambda: (0, 0)),
    ],
    out_specs=[
        pl.BlockSpec((1, _W), lambda: (0, 0)),
        pl.BlockSpec((D, K), lambda: (0, 0)),
    ],
    out_shape=[
        jax.ShapeDtypeStruct((1, _W), jnp.int32),
        jax.ShapeDtypeStruct((D, K), jnp.float32),
    ],
)


@functools.cache
def _make_sc_gather():
    mesh = plsc.VectorSubcoreMesh(core_axis_name="c", subcore_axis_name="s")

    tblk = 128
    drows = D // 2

    @functools.partial(
        pl.kernel,
        mesh=mesh,
        compiler_params=pltpu.CompilerParams(needs_layout_passes=False),
        out_type=jax.ShapeDtypeStruct((B, D, N), jnp.float32),
        scratch_types=[
            pltpu.VMEM((tblk,), jnp.int32),
            pltpu.VMEM((drows, K), jnp.float32),
            pltpu.VMEM((drows, tblk), jnp.float32),
            pltpu.SemaphoreType.DMA,
            pltpu.SemaphoreType.DMA,
        ],
    )
    def _sc_gather(embt_hbm, idx_hbm, out_hbm, idx_v, embt_v, outt_v, sem, sem2):
        wid = lax.axis_index("s") * _NC + lax.axis_index("c")
        b = wid // (_NW // B)
        r = wid % (_NW // B)
        dh = r // (N // tblk)
        tb = r % (N // tblk)
        cp1 = pltpu.async_copy(embt_hbm.at[pl.ds(dh * drows, drows), :],
                               embt_v, sem)
        cp2 = pltpu.async_copy(idx_hbm.at[pl.ds(b * N + tb * tblk, tblk)],
                               idx_v, sem2)
        cp2.wait()
        cp1.wait()

        @plsc.parallel_loop(0, drows, unroll=8)
        def _row(dl):
            drow = jnp.full((_L,), dl, jnp.int32)
            for g in range(tblk // _L):
                tok = idx_v[pl.ds(g * _L, _L)]
                outt_v[dl, pl.ds(g * _L, _L)] = plsc.load_gather(
                    embt_v, [drow, tok])
        pltpu.sync_copy(
            outt_v, out_hbm.at[b, pl.ds(dh * drows, drows),
                               pl.ds(tb * tblk, tblk)])

    return _sc_gather


def kernel(z, W, emb):
    idx, embt = _tc_call(z, W, emb)
    return _make_sc_gather()(embt, idx.reshape(_TOK))


# speedup vs baseline: 1.2697x; 1.0547x over previous
import functools

import jax
import jax.numpy as jnp
from jax import lax
from jax.experimental import pallas as pl
from jax.experimental.pallas import tpu as pltpu
from jax.experimental.pallas import tpu_sc as plsc

B, C_IN, N = 2, 192, 1024
D, K = 64, 512

_NC, _NS, _L = 2, 16, 16
_NW = _NC * _NS
_TOK = B * N


_W = B * N


def _tc_body(z_ref, w_ref, emb_ref, idx_ref, embt_ref):
    w = w_ref[...]
    emb = emb_ref[...]
    hi = lax.Precision.HIGHEST
    wb = w.astype(jnp.bfloat16)
    zb = jnp.concatenate([z_ref[0], z_ref[1]], axis=1)
    ze = jnp.dot(wb, zb.astype(jnp.bfloat16),
                 preferred_element_type=jnp.float32)
    scores = jnp.dot(emb, ze, preferred_element_type=jnp.float32,
                     precision=hi)
    esq2 = 0.5 * jnp.sum(emb * emb, axis=1, keepdims=True)
    dist = esq2 - scores

    iota = lax.broadcasted_iota(jnp.int32, (K, _W), 0)
    i1 = jnp.argmin(dist, axis=0, keepdims=True).astype(jnp.int32)
    dist2 = jnp.where(iota == i1, jnp.float32(jnp.inf), dist)
    i2 = jnp.argmin(dist2, axis=0, keepdims=True).astype(jnp.int32)

    oh1 = (iota == i1).astype(jnp.bfloat16)
    oh2 = (iota == i2).astype(jnp.bfloat16)
    dn = (((0,), (0,)), ((), ()))
    bh = emb.astype(jnp.bfloat16)
    r1 = emb - bh.astype(jnp.float32)
    b1 = r1.astype(jnp.bfloat16)
    b2 = (r1 - b1.astype(jnp.float32)).astype(jnp.bfloat16)

    def _sel(oh):
        p0 = lax.dot_general(bh, oh, dn, preferred_element_type=jnp.float32)
        p1 = lax.dot_general(b1, oh, dn, preferred_element_type=jnp.float32)
        p2 = lax.dot_general(b2, oh, dn, preferred_element_type=jnp.float32)
        return (p0 + p1) + p2

    e1 = _sel(oh1)
    e2 = _sel(oh2)
    d1 = jnp.sum((ze - e1) ** 2, axis=0, keepdims=True)
    d2 = jnp.sum((ze - e2) ** 2, axis=0, keepdims=True)
    pick2 = (d2 < d1) | ((d2 == d1) & (i2 < i1))
    idx_ref[...] = jnp.where(pick2, i2, i1)

    embt_ref[...] = emb.T


_tc_call = pl.pallas_call(
    _tc_body,
    in_specs=[
        pl.BlockSpec((B, C_IN, N), lambda: (0, 0, 0)),
        pl.BlockSpec((D, C_IN), lambda: (0, 0)),
        pl.BlockSpec((K, D), lambda: (0, 0)),
    ],
    out_specs=[
        pl.BlockSpec((1, _W), lambda: (0, 0)),
        pl.BlockSpec((D, K), lambda: (0, 0)),
    ],
    out_shape=[
        jax.ShapeDtypeStruct((1, _W), jnp.int32),
        jax.ShapeDtypeStruct((D, K), jnp.float32),
    ],
)


@functools.cache
def _make_sc_gather():
    mesh = plsc.VectorSubcoreMesh(core_axis_name="c", subcore_axis_name="s")

    tblk = 512
    drows = D // 8

    @functools.partial(
        pl.kernel,
        mesh=mesh,
        compiler_params=pltpu.CompilerParams(needs_layout_passes=False),
        out_type=jax.ShapeDtypeStruct((B, D, N), jnp.float32),
        scratch_types=[
            pltpu.VMEM((tblk,), jnp.int32),
            pltpu.VMEM((drows, K), jnp.float32),
            pltpu.VMEM((drows, tblk), jnp.float32),
            pltpu.SemaphoreType.DMA,
            pltpu.SemaphoreType.DMA,
        ],
    )
    def _sc_gather(embt_hbm, idx_hbm, out_hbm, idx_v, embt_v, outt_v, sem, sem2):
        wid = lax.axis_index("s") * _NC + lax.axis_index("c")
        b = wid // (_NW // B)
        r = wid % (_NW // B)
        dh = r // (N // tblk)
        tb = r % (N // tblk)
        cp1 = pltpu.async_copy(embt_hbm.at[pl.ds(dh * drows, drows), :],
                               embt_v, sem)
        cp2 = pltpu.async_copy(idx_hbm.at[pl.ds(b * N + tb * tblk, tblk)],
                               idx_v, sem2)
        cp2.wait()
        cp1.wait()

        @plsc.parallel_loop(0, drows, unroll=8)
        def _row(dl):
            drow = jnp.full((_L,), dl, jnp.int32)
            for g in range(tblk // _L):
                tok = idx_v[pl.ds(g * _L, _L)]
                outt_v[dl, pl.ds(g * _L, _L)] = plsc.load_gather(
                    embt_v, [drow, tok])
        pltpu.sync_copy(
            outt_v, out_hbm.at[b, pl.ds(dh * drows, drows),
                               pl.ds(tb * tblk, tblk)])

    return _sc_gather


def kernel(z, W, emb):
    idx, embt = _tc_call(z, W, emb)
    return _make_sc_gather()(embt, idx.reshape(_TOK))
